# baseline (speedup 1.0000x reference)
import functools

import jax
import jax.numpy as jnp
from jax import lax
from jax.experimental import pallas as pl
from jax.experimental.pallas import tpu as pltpu
from jax.experimental.pallas import tpu_sc as plsc

_B = 4096
_H = 200
_D = 64
_O = 5

_NBUF = 4


def _sc_pool(idx_t, embed_table):
  info = plsc.get_sparse_core_info()
  nc, ns = info.num_cores, info.num_subcores
  nw = nc * ns
  b_per_w = _B // nw

  mesh = plsc.VectorSubcoreMesh(core_axis_name="c", subcore_axis_name="s")

  @functools.partial(
      pl.kernel,
      out_type=jax.ShapeDtypeStruct((_B, _D), jnp.float32),
      mesh=mesh,
      scratch_types=[
          pltpu.VMEM((_H, b_per_w), jnp.int32),
          pltpu.VMEM((_NBUF, b_per_w, _D), jnp.float32),
          pltpu.VMEM((b_per_w, _D), jnp.float32),
      ] + [pltpu.SemaphoreType.DMA] * _NBUF,
      compiler_params=pltpu.CompilerParams(use_tc_tiling_on_sc=False),
  )
  def k(idx_hbm, table_hbm, out_hbm, idx_v, rows_v, pooled_v, *sems):
    wid = lax.axis_index("s") * nc + lax.axis_index("c")
    base = wid * b_per_w
    pltpu.sync_copy(idx_hbm.at[:, pl.ds(base, b_per_w)], idx_v)

    zero = jnp.zeros((16,), jnp.float32)

    @pl.loop(0, b_per_w)
    def _(j):
      for c in range(_D // 16):
        pooled_v[j, pl.ds(c * 16, 16)] = zero

    def gather_desc(l, nb):
      return pltpu.make_async_copy(
          table_hbm.at[idx_v.at[l]], rows_v.at[nb], sems[nb])

    def accum(nb):
      @pl.loop(0, b_per_w, unroll=4)
      def _(j):
        for c in range(_D // 16):
          plsc.addupdate(pooled_v.at[j, pl.ds(c * 16, 16)],
                         rows_v[nb, j, pl.ds(c * 16, 16)])

    for nb in range(_NBUF):
      gather_desc(nb, nb).start()

    @pl.loop(0, _H, step=_NBUF)
    def _(l):
      for nb in range(_NBUF):
        pos = l + nb
        gather_desc(pos, nb).wait()
        accum(nb)

        @pl.when(pos + _NBUF < _H)
        def _():
          gather_desc(pos + _NBUF, nb).start()

    pltpu.sync_copy(pooled_v, out_hbm.at[pl.ds(base, b_per_w)])

  return k(idx_t, embed_table)


def _tc_linear(pooled, bias2, W, b2):
  def body(pooled_ref, bias_ref, w_ref, b_ref, out_ref):
    x = pooled_ref[...] + bias_ref[...]
    out_ref[...] = (
        jnp.dot(x, w_ref[...], preferred_element_type=jnp.float32)
        + b_ref[...]
    )

  return pl.pallas_call(
      body,
      out_shape=jax.ShapeDtypeStruct((_B, _O), jnp.float32),
  )(pooled, bias2, W, b2)


def kernel(inputs, embed_table, bias, W, b):
  pooled = _sc_pool(inputs.astype(jnp.int32).T, embed_table)
  return _tc_linear(pooled, bias.reshape(1, _D), W, b.reshape(1, _O))
